# baseline (speedup 1.0000x reference)
import functools

import jax
import jax.numpy as jnp
from jax import lax
from jax.experimental import pallas as pl
from jax.experimental.pallas import tpu as pltpu
from jax.experimental.pallas import tpu_sc as plsc

NN = 50000
HH = 50000
EE = 800000
GG = 64
NIN = 92
HD = 64
HEA = 35
HOUTD = 128
LL = 3

NC, NSUB = 2, 16
BCH = 128
K = 392
EPAD = NSUB * K * BCH
ROWS = 51200
ZR = ROWS // NSUB
QL = 8
QP = 8
PADROW = 50000

F32 = jnp.float32
I32 = jnp.int32


def _sc_mesh():
    return plsc.VectorSubcoreMesh(
        core_axis_name="c", subcore_axis_name="s",
        num_cores=NC, num_subcores=NSUB)


def _zero_acc(z_h, acc, s):
    pltpu.sync_copy(z_h, acc.at[pl.ds(s * ZR, ZR)])


def _edge_loop(c, t0, t1, gv, sv, bufa, bufb, sema, semb, acc):
    def sg(j, buf, sem):
        @pl.when(c == 0)
        def _():
            pltpu.async_copy(t0.at[gv.at[j]], buf, sem)

        @pl.when(c == 1)
        def _():
            pltpu.async_copy(t1.at[gv.at[j]], buf, sem)

    def wg(buf, sem):
        pltpu.make_async_copy(t0.at[pl.ds(0, BCH)], buf, sem).wait()

    sg(0, bufa, sema)

    def body(j2, _):
        j = j2 * 2
        sg(j + 1, bufb, semb)
        wg(bufa, sema)
        pltpu.sync_copy(bufa, acc.at[sv.at[j]], add=True)

        @pl.when(j + 2 < K)
        def _():
            sg(j + 2, bufa, sema)

        wg(bufb, semb)
        pltpu.sync_copy(bufb, acc.at[sv.at[j + 1]], add=True)
        return 0

    lax.fori_loop(0, K // 2, body, 0)


def _scatter_only_loop(sv, buf, acc):
    def body(j, _):
        pltpu.sync_copy(buf, acc.at[sv.at[j]], add=True)
        return 0

    lax.fori_loop(0, K, body, 0)


def _raw_out(c, s, acc, o0, o1):
    @pl.when(c == 0)
    def _():
        pltpu.sync_copy(acc.at[pl.ds(s * ZR, ZR)], o0.at[pl.ds(s * ZR, ZR)])

    @pl.when(c == 1)
    def _():
        pltpu.sync_copy(acc.at[pl.ds(s * ZR, ZR)], o1.at[pl.ds(s * ZR, ZR)])


@functools.partial(
    pl.kernel,
    out_type=tuple(jax.ShapeDtypeStruct((ROWS, QP), F32) for _ in range(8)),
    mesh=_sc_mesh(),
    compiler_params=pltpu.CompilerParams(use_tc_tiling_on_sc=False),
    scratch_types=[
        pltpu.VMEM((K, BCH), I32),
        pltpu.VMEM((K, BCH), I32),
        pltpu.VMEM((BCH, QP), F32),
        pltpu.VMEM((BCH, QP), F32),
        pltpu.VMEM((BCH, QP), F32),
        pltpu.VMEM_SHARED((ROWS, QP), F32),
        pltpu.SemaphoreType.DMA,
        pltpu.SemaphoreType.DMA,
    ],
)
def _sc_prep(src_h, hid_h, a0_h, a1_h, a2_h, a3_h, a4_h, a5_h, ones_h, z_h,
             oa0, oa1, oa2, oa3, oa4, oa5, ocn, och,
             srcv, hidv, bufa, bufb, onesb, acc, sema, semb):
    c = lax.axis_index("c")
    s = lax.axis_index("s")

    pltpu.sync_copy(src_h.at[s], srcv)
    pltpu.sync_copy(hid_h.at[s], hidv)
    pltpu.sync_copy(ones_h, onesb)

    ains = (a0_h, a1_h, a2_h, a3_h, a4_h, a5_h)
    aouts = (oa0, oa1, oa2, oa3, oa4, oa5)

    for p in range(3):
        _zero_acc(z_h, acc, s)
        plsc.subcore_barrier()
        _edge_loop(c, ains[2 * p], ains[2 * p + 1], hidv, srcv,
                   bufa, bufb, sema, semb, acc)
        plsc.subcore_barrier()
        _raw_out(c, s, acc, aouts[2 * p], aouts[2 * p + 1])

    _zero_acc(z_h, acc, s)
    plsc.subcore_barrier()

    @pl.when(c == 0)
    def _():
        _scatter_only_loop(srcv, onesb, acc)

    @pl.when(c == 1)
    def _():
        _scatter_only_loop(hidv, onesb, acc)

    plsc.subcore_barrier()
    _raw_out(c, s, acc, ocn, och)


@functools.partial(
    pl.kernel,
    out_type=tuple(jax.ShapeDtypeStruct((ROWS, QL), F32) for _ in range(8))
    + (jax.ShapeDtypeStruct((ZR, QL), F32),),
    mesh=_sc_mesh(),
    compiler_params=pltpu.CompilerParams(use_tc_tiling_on_sc=False),
    scratch_types=[
        pltpu.VMEM((K, BCH), I32),
        pltpu.VMEM((K, BCH), I32),
        pltpu.VMEM((BCH, QL), F32),
        pltpu.VMEM((BCH, QL), F32),
        pltpu.VMEM_SHARED((ROWS, QL), F32),
        pltpu.SemaphoreType.DMA,
        pltpu.SemaphoreType.DMA,
    ],
)
def _sc_seg(gid_h, sid_h, t0_h, t1_h, t2_h, t3_h, t4_h, t5_h, t6_h, t7_h,
            z_h,
            o0, o1, o2, o3, o4, o5, o6, o7, o_z,
            gidv, sidv, bufa, bufb, acc, sema, semb):
    c = lax.axis_index("c")
    s = lax.axis_index("s")

    @pl.when(jnp.logical_and(c == 0, s == 0))
    def _():
        pltpu.sync_copy(z_h, o_z)

    pltpu.sync_copy(gid_h.at[s], gidv)
    pltpu.sync_copy(sid_h.at[s], sidv)

    tins = (t0_h, t1_h, t2_h, t3_h, t4_h, t5_h, t6_h, t7_h)
    outs = (o0, o1, o2, o3, o4, o5, o6, o7)

    for p in range(4):
        _zero_acc(z_h, acc, s)
        plsc.subcore_barrier()
        _edge_loop(c, tins[2 * p], tins[2 * p + 1], gidv, sidv,
                   bufa, bufb, sema, semb, acc)
        plsc.subcore_barrier()
        _raw_out(c, s, acc, outs[2 * p], outs[2 * p + 1])
        plsc.subcore_barrier()


BLK = 512
GRID = ROWS // BLK


def _tc_embed_body(x_ref, w_ref, *h_refs):
    h = jnp.dot(x_ref[...], w_ref[...], preferred_element_type=F32)
    for i in range(8):
        h_refs[i][...] = h[:, 8 * i:8 * i + 8]


def _tc_embed(x_aug, w_aug):
    spec8 = pl.BlockSpec((BLK, QL), lambda i: (i, 0))
    return pl.pallas_call(
        _tc_embed_body,
        grid=(GRID,),
        in_specs=[pl.BlockSpec((BLK, 128), lambda i: (i, 0)),
                  pl.BlockSpec((128, HD), lambda i: (0, 0))],
        out_specs=[spec8] * 8,
        out_shape=[jax.ShapeDtypeStruct((ROWS, QL), F32)] * 8,
    )(x_aug, w_aug)


def _softplus(x):
    return jnp.maximum(x, 0.0) + jnp.log1p(jnp.exp(-jnp.abs(x)))


RIL = ROWS // 16
BIL = RIL // 10


def _tc_scale_body(*refs):
    s_refs = refs[0:8]
    inv_ref = refs[8]
    o_refs = refs[9:17]
    inv = inv_ref[...]
    for i in range(8):
        o_refs[i][...] = s_refs[i][...] * inv


def _tc_scale(sums, invh_il):
    spec = pl.BlockSpec((BIL, 128), lambda i: (i, 0))
    il = [s.reshape(RIL, 128) for s in sums]
    outs = pl.pallas_call(
        _tc_scale_body,
        grid=(RIL // BIL,),
        in_specs=[spec] * 9,
        out_specs=[spec] * 8,
        out_shape=[jax.ShapeDtypeStruct((RIL, 128), F32)] * 8,
    )(*il, invh_il)
    return tuple(o.reshape(ROWS, QL) for o in outs)


def _tc_layer_body(*refs):
    h_refs = refs[0:8]
    a_refs = refs[8:11]
    g_refs = refs[11:19]
    c_ref, w_ref, b_ref = refs[19:22]
    n_refs = refs[22:30]
    cnt = c_ref[...][:, 0:1]
    inv = 1.0 / jnp.maximum(cnt, 1.0)
    m = (cnt > 0.0).astype(F32)
    hq = [r[...] for r in h_refs]
    h = jnp.concatenate(hq, axis=1)
    z = jnp.concatenate(
        [q * m for q in hq]
        + [a[...] * inv for a in a_refs]
        + [g[...] * inv for g in g_refs],
        axis=1)
    o = jnp.dot(z, w_ref[...], preferred_element_type=F32) + b_ref[...]
    zf = o[:, :HD]
    zc = o[:, HD:]
    out = jax.nn.sigmoid(zf) * _softplus(zc)
    hn = _softplus(out + h)
    for i in range(8):
        n_refs[i][...] = hn[:, 8 * i:8 * i + 8]


def _tc_layer(hq, aq, gq, cnt_n, w_stk, b_stk):
    spec8 = pl.BlockSpec((BLK, QL), lambda i: (i, 0))
    specq = pl.BlockSpec((BLK, 16), lambda i: (i, 0))
    specc = pl.BlockSpec((BLK, QP), lambda i: (i, 0))
    return pl.pallas_call(
        _tc_layer_body,
        grid=(GRID,),
        in_specs=[spec8] * 8 + [specq] * 3 + [spec8] * 8 + [
            specc,
            pl.BlockSpec((176, 128), lambda i: (0, 0)),
            pl.BlockSpec((1, 128), lambda i: (0, 0))],
        out_specs=[spec8] * 8,
        out_shape=[jax.ShapeDtypeStruct((ROWS, QL), F32)] * 8,
    )(*hq, *aq, *gq, cnt_n, w_stk, b_stk)


def _tc_out_body(h0_ref, h1_ref, h2_ref, h3_ref, h4_ref, h5_ref, h6_ref,
                 h7_ref, b_ref, wp_ref, bp_ref, wo_ref, bo_ref, o_ref,
                 acc, cacc):
    i = pl.program_id(0)

    @pl.when(i == 0)
    def _():
        acc[...] = jnp.zeros_like(acc)
        cacc[...] = jnp.zeros_like(cacc)

    bb = b_ref[0]
    gid = lax.broadcasted_iota(I32, (GG, BLK), 0)
    maskf = (jnp.broadcast_to(bb, (GG, BLK)) == gid).astype(F32)
    h = jnp.concatenate(
        [h0_ref[...], h1_ref[...], h2_ref[...], h3_ref[...],
         h4_ref[...], h5_ref[...], h6_ref[...], h7_ref[...]], axis=1)
    acc[...] += jnp.dot(maskf, h, preferred_element_type=F32)
    cacc[...] += jnp.broadcast_to(
        jnp.sum(maskf, axis=1, keepdims=True), (GG, 128))

    @pl.when(i == GRID - 1)
    def _():
        cnt = cacc[:, 0:1]
        gm = acc[...] / jnp.maximum(cnt, 1.0)
        p = _softplus(jnp.dot(gm, wp_ref[...], preferred_element_type=F32)
                      + bp_ref[...])
        res = jnp.dot(p, wo_ref[...], preferred_element_type=F32) + bo_ref[...]
        o_ref[...] = res[:, 0:1]


def _tc_out(hq, batch3d, wp, bp, wo8, bo8):
    spec8 = pl.BlockSpec((BLK, QL), lambda i: (i, 0))
    return pl.pallas_call(
        _tc_out_body,
        grid=(GRID,),
        in_specs=[spec8] * 8 + [
            pl.BlockSpec((1, 1, BLK), lambda i: (i, 0, 0)),
            pl.BlockSpec((HD, HOUTD), lambda i: (0, 0)),
            pl.BlockSpec((1, HOUTD), lambda i: (0, 0)),
            pl.BlockSpec((HOUTD, 8), lambda i: (0, 0)),
            pl.BlockSpec((1, 8), lambda i: (0, 0))],
        out_specs=pl.BlockSpec((GG, 1), lambda i: (0, 0)),
        out_shape=jax.ShapeDtypeStruct((GG, 1), F32),
        scratch_shapes=[pltpu.VMEM((GG, HD), F32),
                        pltpu.VMEM((GG, 128), F32)],
    )(*hq, batch3d, wp, bp, wo8, bo8)


def kernel(x, hyperedge_index, hyperedge_attr, batch, W_embed, b_embed,
           Wf, bf, Wc, bc, W_proj, b_proj, W_out, b_out):
    src = hyperedge_index[0].astype(I32)
    hid = hyperedge_index[1].astype(I32)
    padv = jnp.full((EPAD - EE,), PADROW, I32)
    src3 = jnp.concatenate([src, padv]).reshape(NSUB, K, BCH)
    hid3 = jnp.concatenate([hid, padv]).reshape(NSUB, K, BCH)

    zrows_p = jnp.zeros((ZR, QP), F32)
    zrows = jnp.zeros((ZR, QL), F32)
    ones_s = jnp.ones((BCH, QP), F32)

    attr_p = jnp.pad(hyperedge_attr, ((0, ROWS - HH), (0, 48 - HEA)))
    aslices = [attr_p[:, 8 * i:8 * i + 8] for i in range(6)]
    prep = _sc_prep(src3, hid3, *aslices, ones_s, zrows_p)
    a48 = jnp.concatenate(prep[:6], axis=1)
    cnt_n = prep[6]
    cnt_h = prep[7]
    aq = (a48[:, 0:16], a48[:, 16:32], a48[:, 32:48])

    x_aug = jnp.pad(x, ((0, ROWS - NN), (0, 128 - NIN)))
    x_aug = x_aug.at[:, NIN].set(1.0)
    w_aug = jnp.zeros((128, HD), F32).at[:NIN].set(W_embed).at[NIN].set(b_embed)
    hq = tuple(_tc_embed(x_aug, w_aug))

    def stack_w(wf_i, wc_i):
        return jnp.concatenate([
            jnp.concatenate([wf_i[:96], wc_i[:96]], 1),
            jnp.pad(jnp.concatenate([wf_i[96:99], wc_i[96:99]], 1),
                    ((0, 13), (0, 0))),
            jnp.concatenate([wf_i[99:], wc_i[99:]], 1),
        ], 0)

    ws = jnp.stack([stack_w(Wf[i], Wc[i]) for i in range(LL)])
    bs = jnp.stack([jnp.concatenate([bf[i], bc[i]])[None, :]
                    for i in range(LL)])

    invh_il = jnp.broadcast_to(
        1.0 / jnp.maximum(cnt_h[:, 0:1], 1.0), (ROWS, QL)).reshape(RIL, 128)

    def half_step(t, carry):
        hq_c, tab, gids, sids, ztok = carry
        even = (t % 2) == 0
        outs = _sc_seg(gids, sids, *tab, ztok)
        sums = tuple(outs[:8])

        def do_scale(_):
            return hq_c, _tc_scale(sums, invh_il)

        def do_gate(_):
            w_stk = lax.dynamic_index_in_dim(ws, t // 2, keepdims=False)
            b_stk = lax.dynamic_index_in_dim(bs, t // 2, keepdims=False)
            hn = tuple(_tc_layer(hq_c, aq, sums, cnt_n, w_stk, b_stk))
            return hn, hn

        hq_n, tab_n = lax.cond(even, do_scale, do_gate, 0)
        return hq_n, tab_n, sids, gids, outs[8]

    nsteps = lax.optimization_barrier(jnp.int32(2 * LL))
    hq, _, _, _, _ = lax.fori_loop(
        0, nsteps, half_step, (hq, hq, src3, hid3, zrows))

    batch3d = jnp.pad(batch.astype(I32), (0, ROWS - NN),
                      constant_values=GG).reshape(GRID, 1, BLK)
    wo8 = jnp.pad(W_out, ((0, 0), (0, 7)))
    bo8 = jnp.pad(b_out, (0, 7))[None, :]
    return _tc_out(hq, batch3d, W_proj, b_proj[None, :], wo8, bo8)


# speedup vs baseline: 1.0017x; 1.0017x over previous
import functools

import jax
import jax.numpy as jnp
from jax import lax
from jax.experimental import pallas as pl
from jax.experimental.pallas import tpu as pltpu
from jax.experimental.pallas import tpu_sc as plsc

NN = 50000
HH = 50000
EE = 800000
GG = 64
NIN = 92
HD = 64
HEA = 35
HOUTD = 128
LL = 3

NC, NSUB = 2, 16
BCH = 128
K = 392
EPAD = NSUB * K * BCH
ROWS = 51200
ZR = ROWS // NSUB
QL = 8
QP = 8
PADROW = 50000

F32 = jnp.float32
I32 = jnp.int32


def _sc_mesh():
    return plsc.VectorSubcoreMesh(
        core_axis_name="c", subcore_axis_name="s",
        num_cores=NC, num_subcores=NSUB)


def _zero_acc(z_h, acc, s):
    pltpu.sync_copy(z_h, acc.at[pl.ds(s * ZR, ZR)])


def _edge_loop(c, t0, t1, gv, sv, bufa, bufb, sema, semb, acc):
    def sg(j, buf, sem):
        @pl.when(c == 0)
        def _():
            pltpu.async_copy(t0.at[gv.at[j]], buf, sem)

        @pl.when(c == 1)
        def _():
            pltpu.async_copy(t1.at[gv.at[j]], buf, sem)

    def wg(buf, sem):
        pltpu.make_async_copy(t0.at[pl.ds(0, BCH)], buf, sem).wait()

    sg(0, bufa, sema)

    def body(j2, _):
        j = j2 * 2
        sg(j + 1, bufb, semb)
        wg(bufa, sema)
        pltpu.sync_copy(bufa, acc.at[sv.at[j]], add=True)

        @pl.when(j + 2 < K)
        def _():
            sg(j + 2, bufa, sema)

        wg(bufb, semb)
        pltpu.sync_copy(bufb, acc.at[sv.at[j + 1]], add=True)
        return 0

    lax.fori_loop(0, K // 2, body, 0)


def _scatter_only_loop(sv, buf, acc):
    def body(j, _):
        pltpu.sync_copy(buf, acc.at[sv.at[j]], add=True)
        return 0

    lax.fori_loop(0, K, body, 0)


def _raw_out(c, s, acc, o0, o1):
    @pl.when(c == 0)
    def _():
        pltpu.sync_copy(acc.at[pl.ds(s * ZR, ZR)], o0.at[pl.ds(s * ZR, ZR)])

    @pl.when(c == 1)
    def _():
        pltpu.sync_copy(acc.at[pl.ds(s * ZR, ZR)], o1.at[pl.ds(s * ZR, ZR)])


@functools.partial(
    pl.kernel,
    out_type=tuple(jax.ShapeDtypeStruct((ROWS, QP), F32) for _ in range(8)),
    mesh=_sc_mesh(),
    compiler_params=pltpu.CompilerParams(use_tc_tiling_on_sc=False),
    scratch_types=[
        pltpu.VMEM((K, BCH), I32),
        pltpu.VMEM((K, BCH), I32),
        pltpu.VMEM((BCH, QP), F32),
        pltpu.VMEM((BCH, QP), F32),
        pltpu.VMEM((BCH, QP), F32),
        pltpu.VMEM_SHARED((ROWS, QP), F32),
        pltpu.SemaphoreType.DMA,
        pltpu.SemaphoreType.DMA,
    ],
)
def _sc_prep(src_h, hid_h, a0_h, a1_h, a2_h, a3_h, a4_h, a5_h, ones_h, z_h,
             oa0, oa1, oa2, oa3, oa4, oa5, ocn, och,
             srcv, hidv, bufa, bufb, onesb, acc, sema, semb):
    c = lax.axis_index("c")
    s = lax.axis_index("s")

    pltpu.sync_copy(src_h.at[s], srcv)
    pltpu.sync_copy(hid_h.at[s], hidv)
    pltpu.sync_copy(ones_h, onesb)

    ains = (a0_h, a1_h, a2_h, a3_h, a4_h, a5_h)
    aouts = (oa0, oa1, oa2, oa3, oa4, oa5)

    for p in range(3):
        _zero_acc(z_h, acc, s)
        plsc.subcore_barrier()
        _edge_loop(c, ains[2 * p], ains[2 * p + 1], hidv, srcv,
                   bufa, bufb, sema, semb, acc)
        plsc.subcore_barrier()
        _raw_out(c, s, acc, aouts[2 * p], aouts[2 * p + 1])

    _zero_acc(z_h, acc, s)
    plsc.subcore_barrier()

    @pl.when(c == 0)
    def _():
        _scatter_only_loop(srcv, onesb, acc)

    @pl.when(c == 1)
    def _():
        _scatter_only_loop(hidv, onesb, acc)

    plsc.subcore_barrier()
    _raw_out(c, s, acc, ocn, och)


@functools.partial(
    pl.kernel,
    out_type=tuple(jax.ShapeDtypeStruct((ROWS, QL), F32) for _ in range(8))
    + (jax.ShapeDtypeStruct((ZR, QL), F32),),
    mesh=_sc_mesh(),
    compiler_params=pltpu.CompilerParams(use_tc_tiling_on_sc=False),
    scratch_types=[
        pltpu.VMEM((K, BCH), I32),
        pltpu.VMEM((K, BCH), I32),
        pltpu.VMEM((BCH, QL), F32),
        pltpu.VMEM((BCH, QL), F32),
        pltpu.VMEM_SHARED((ROWS, QL), F32),
        pltpu.SemaphoreType.DMA,
        pltpu.SemaphoreType.DMA,
    ],
)
def _sc_seg(gid_h, sid_h, t0_h, t1_h, t2_h, t3_h, t4_h, t5_h, t6_h, t7_h,
            z_h,
            o0, o1, o2, o3, o4, o5, o6, o7, o_z,
            gidv, sidv, bufa, bufb, acc, sema, semb):
    c = lax.axis_index("c")
    s = lax.axis_index("s")

    @pl.when(jnp.logical_and(c == 0, s == 0))
    def _():
        pltpu.sync_copy(z_h, o_z)

    pltpu.sync_copy(gid_h.at[s], gidv)
    pltpu.sync_copy(sid_h.at[s], sidv)

    tins = (t0_h, t1_h, t2_h, t3_h, t4_h, t5_h, t6_h, t7_h)
    outs = (o0, o1, o2, o3, o4, o5, o6, o7)

    for p in range(4):
        _zero_acc(z_h, acc, s)
        plsc.subcore_barrier()
        _edge_loop(c, tins[2 * p], tins[2 * p + 1], gidv, sidv,
                   bufa, bufb, sema, semb, acc)
        plsc.subcore_barrier()
        _raw_out(c, s, acc, outs[2 * p], outs[2 * p + 1])
        plsc.subcore_barrier()


BLK = 512
GRID = ROWS // BLK


def _tc_embed_body(x_ref, w_ref, *h_refs):
    h = jnp.dot(x_ref[...], w_ref[...], preferred_element_type=F32)
    for i in range(8):
        h_refs[i][...] = h[:, 8 * i:8 * i + 8]


def _tc_embed(x_aug, w_aug):
    spec8 = pl.BlockSpec((BLK, QL), lambda i: (i, 0))
    return pl.pallas_call(
        _tc_embed_body,
        grid=(GRID,),
        in_specs=[pl.BlockSpec((BLK, 128), lambda i: (i, 0)),
                  pl.BlockSpec((128, HD), lambda i: (0, 0))],
        out_specs=[spec8] * 8,
        out_shape=[jax.ShapeDtypeStruct((ROWS, QL), F32)] * 8,
    )(x_aug, w_aug)


def _softplus(x):
    return jnp.maximum(x, 0.0) + jnp.log1p(jnp.exp(-jnp.abs(x)))


RIL = ROWS // 16
BIL = RIL // 10


def _tc_scale_body(*refs):
    s_refs = refs[0:8]
    inv_ref = refs[8]
    o_refs = refs[9:17]
    inv = inv_ref[...]
    for i in range(8):
        o_refs[i][...] = s_refs[i][...] * inv


def _tc_scale(sums, invh_il):
    spec = pl.BlockSpec((BIL, 128), lambda i: (i, 0))
    il = [s.reshape(RIL, 128) for s in sums]
    outs = pl.pallas_call(
        _tc_scale_body,
        grid=(RIL // BIL,),
        in_specs=[spec] * 9,
        out_specs=[spec] * 8,
        out_shape=[jax.ShapeDtypeStruct((RIL, 128), F32)] * 8,
    )(*il, invh_il)
    return tuple(o.reshape(ROWS, QL) for o in outs)


def _tc_layer_body(*refs):
    h_refs = refs[0:8]
    a_refs = refs[8:11]
    g_refs = refs[11:19]
    c_ref, w_ref, b_ref = refs[19:22]
    n_refs = refs[22:30]
    cnt = c_ref[...][:, 0:1]
    inv = 1.0 / jnp.maximum(cnt, 1.0)
    m = (cnt > 0.0).astype(F32)
    hq = [r[...] for r in h_refs]
    h = jnp.concatenate(hq, axis=1)
    z = jnp.concatenate(
        [q * m for q in hq]
        + [a[...] * inv for a in a_refs]
        + [g[...] * inv for g in g_refs],
        axis=1)
    o = jnp.dot(z, w_ref[...], preferred_element_type=F32) + b_ref[...]
    zf = o[:, :HD]
    zc = o[:, HD:]
    out = jax.nn.sigmoid(zf) * _softplus(zc)
    hn = _softplus(out + h)
    for i in range(8):
        n_refs[i][...] = hn[:, 8 * i:8 * i + 8]


def _tc_layer(hq, aq, gq, cnt_n, w_stk, b_stk):
    spec8 = pl.BlockSpec((BLK, QL), lambda i: (i, 0))
    specq = pl.BlockSpec((BLK, 16), lambda i: (i, 0))
    specc = pl.BlockSpec((BLK, QP), lambda i: (i, 0))
    return pl.pallas_call(
        _tc_layer_body,
        grid=(GRID,),
        in_specs=[spec8] * 8 + [specq] * 3 + [spec8] * 8 + [
            specc,
            pl.BlockSpec((176, 128), lambda i: (0, 0)),
            pl.BlockSpec((1, 128), lambda i: (0, 0))],
        out_specs=[spec8] * 8,
        out_shape=[jax.ShapeDtypeStruct((ROWS, QL), F32)] * 8,
    )(*hq, *aq, *gq, cnt_n, w_stk, b_stk)


def _tc_out_body(h0_ref, h1_ref, h2_ref, h3_ref, h4_ref, h5_ref, h6_ref,
                 h7_ref, b_ref, wp_ref, bp_ref, wo_ref, bo_ref, o_ref,
                 acc, cacc):
    i = pl.program_id(0)

    @pl.when(i == 0)
    def _():
        acc[...] = jnp.zeros_like(acc)
        cacc[...] = jnp.zeros_like(cacc)

    bb = b_ref[0]
    gid = lax.broadcasted_iota(I32, (GG, BLK), 0)
    maskf = (jnp.broadcast_to(bb, (GG, BLK)) == gid).astype(F32)
    h = jnp.concatenate(
        [h0_ref[...], h1_ref[...], h2_ref[...], h3_ref[...],
         h4_ref[...], h5_ref[...], h6_ref[...], h7_ref[...]], axis=1)
    acc[...] += jnp.dot(maskf, h, preferred_element_type=F32)
    cacc[...] += jnp.broadcast_to(
        jnp.sum(maskf, axis=1, keepdims=True), (GG, 128))

    @pl.when(i == GRID - 1)
    def _():
        cnt = cacc[:, 0:1]
        gm = acc[...] / jnp.maximum(cnt, 1.0)
        p = _softplus(jnp.dot(gm, wp_ref[...], preferred_element_type=F32)
                      + bp_ref[...])
        res = jnp.dot(p, wo_ref[...], preferred_element_type=F32) + bo_ref[...]
        o_ref[...] = res[:, 0:1]


def _tc_out(hq, batch3d, wp, bp, wo8, bo8):
    spec8 = pl.BlockSpec((BLK, QL), lambda i: (i, 0))
    return pl.pallas_call(
        _tc_out_body,
        grid=(GRID,),
        in_specs=[spec8] * 8 + [
            pl.BlockSpec((1, 1, BLK), lambda i: (i, 0, 0)),
            pl.BlockSpec((HD, HOUTD), lambda i: (0, 0)),
            pl.BlockSpec((1, HOUTD), lambda i: (0, 0)),
            pl.BlockSpec((HOUTD, 8), lambda i: (0, 0)),
            pl.BlockSpec((1, 8), lambda i: (0, 0))],
        out_specs=pl.BlockSpec((GG, 1), lambda i: (0, 0)),
        out_shape=jax.ShapeDtypeStruct((GG, 1), F32),
        scratch_shapes=[pltpu.VMEM((GG, HD), F32),
                        pltpu.VMEM((GG, 128), F32)],
    )(*hq, batch3d, wp, bp, wo8, bo8)


def kernel(x, hyperedge_index, hyperedge_attr, batch, W_embed, b_embed,
           Wf, bf, Wc, bc, W_proj, b_proj, W_out, b_out):
    src = hyperedge_index[0].astype(I32)
    hid = hyperedge_index[1].astype(I32)
    padv = jnp.full((EPAD - EE,), PADROW, I32)
    src3 = jnp.concatenate([src, padv]).reshape(NSUB, K, BCH)
    hid3 = jnp.concatenate([hid, padv]).reshape(NSUB, K, BCH)

    zrows_p = jnp.zeros((ZR, QP), F32)
    zrows = jnp.zeros((ZR, QL), F32)
    ones_s = jnp.ones((BCH, QP), F32)

    attr_p = jnp.pad(hyperedge_attr, ((0, ROWS - HH), (0, 48 - HEA)))
    aslices = [attr_p[:, 8 * i:8 * i + 8] for i in range(6)]
    prep = _sc_prep(src3, hid3, *aslices, ones_s, zrows_p)
    a48 = jnp.concatenate(prep[:6], axis=1)
    cnt_n = prep[6]
    cnt_h = prep[7]
    aq = (a48[:, 0:16], a48[:, 16:32], a48[:, 32:48])

    x_aug = jnp.pad(x, ((0, ROWS - NN), (0, 128 - NIN)))
    x_aug = x_aug.at[:, NIN].set(1.0)
    w_aug = jnp.zeros((128, HD), F32).at[:NIN].set(W_embed).at[NIN].set(b_embed)
    hq = tuple(_tc_embed(x_aug, w_aug))

    def stack_w(wf_i, wc_i):
        return jnp.concatenate([
            jnp.concatenate([wf_i[:96], wc_i[:96]], 1),
            jnp.pad(jnp.concatenate([wf_i[96:99], wc_i[96:99]], 1),
                    ((0, 13), (0, 0))),
            jnp.concatenate([wf_i[99:], wc_i[99:]], 1),
        ], 0)

    ws = jnp.stack([stack_w(Wf[i], Wc[i]) for i in range(LL)])
    bs = jnp.stack([jnp.concatenate([bf[i], bc[i]])[None, :]
                    for i in range(LL)])

    invh_il = jnp.broadcast_to(
        1.0 / jnp.maximum(cnt_h[:, 0:1], 1.0), (ROWS, QL)).reshape(RIL, 128)

    def half_step(t, carry):
        hq_c, tab, ztok = carry
        even = (t % 2) == 0
        gids = jnp.where(even, src3, hid3)
        sids = jnp.where(even, hid3, src3)
        outs = _sc_seg(gids, sids, *tab, ztok)
        sums = tuple(outs[:8])

        def do_scale(_):
            return hq_c, _tc_scale(sums, invh_il)

        def do_gate(_):
            w_stk = lax.dynamic_index_in_dim(ws, t // 2, keepdims=False)
            b_stk = lax.dynamic_index_in_dim(bs, t // 2, keepdims=False)
            hn = tuple(_tc_layer(hq_c, aq, sums, cnt_n, w_stk, b_stk))
            return hn, hn

        hq_n, tab_n = lax.cond(even, do_scale, do_gate, 0)
        return hq_n, tab_n, outs[8]

    nsteps = lax.optimization_barrier(jnp.int32(2 * LL))
    hq, _, _ = lax.fori_loop(0, nsteps, half_step, (hq, hq, zrows))

    batch3d = jnp.pad(batch.astype(I32), (0, ROWS - NN),
                      constant_values=GG).reshape(GRID, 1, BLK)
    wo8 = jnp.pad(W_out, ((0, 0), (0, 7)))
    bo8 = jnp.pad(b_out, (0, 7))[None, :]
    return _tc_out(hq, batch3d, W_proj, b_proj[None, :], wo8, bo8)


# speedup vs baseline: 1.0502x; 1.0484x over previous
import functools

import jax
import jax.numpy as jnp
from jax import lax
from jax.experimental import pallas as pl
from jax.experimental.pallas import tpu as pltpu
from jax.experimental.pallas import tpu_sc as plsc

NN = 50000
HH = 50000
EE = 800000
GG = 64
NIN = 92
HD = 64
HEA = 35
HOUTD = 128
LL = 3

NC, NSUB = 2, 16
BCH = 128
K = 392
EPAD = NSUB * K * BCH
ROWS = 51200
ZR = ROWS // NSUB
QL = 8
QP = 8
PADROW = 50000

F32 = jnp.float32
I32 = jnp.int32


def _sc_mesh():
    return plsc.VectorSubcoreMesh(
        core_axis_name="c", subcore_axis_name="s",
        num_cores=NC, num_subcores=NSUB)


def _zero_acc(z_h, acc, s):
    pltpu.sync_copy(z_h, acc.at[pl.ds(s * ZR, ZR)])


def _edge_loop(c, t0, t1, gv, sv, bufa, bufb, sema, semb, acc):
    def sg(j, buf, sem):
        @pl.when(c == 0)
        def _():
            pltpu.async_copy(t0.at[gv.at[j]], buf, sem)

        @pl.when(c == 1)
        def _():
            pltpu.async_copy(t1.at[gv.at[j]], buf, sem)

    def wg(buf, sem):
        pltpu.make_async_copy(t0.at[pl.ds(0, BCH)], buf, sem).wait()

    sg(0, bufa, sema)

    def body(j2, _):
        j = j2 * 2
        sg(j + 1, bufb, semb)
        wg(bufa, sema)
        pltpu.sync_copy(bufa, acc.at[sv.at[j]], add=True)

        @pl.when(j + 2 < K)
        def _():
            sg(j + 2, bufa, sema)

        wg(bufb, semb)
        pltpu.sync_copy(bufb, acc.at[sv.at[j + 1]], add=True)
        return 0

    lax.fori_loop(0, K // 2, body, 0)


def _scatter_only_loop(sv, buf, acc):
    def body(j, _):
        pltpu.sync_copy(buf, acc.at[sv.at[j]], add=True)
        return 0

    lax.fori_loop(0, K, body, 0)


def _raw_out(c, s, acc, o0, o1):
    @pl.when(c == 0)
    def _():
        pltpu.sync_copy(acc.at[pl.ds(s * ZR, ZR)], o0.at[pl.ds(s * ZR, ZR)])

    @pl.when(c == 1)
    def _():
        pltpu.sync_copy(acc.at[pl.ds(s * ZR, ZR)], o1.at[pl.ds(s * ZR, ZR)])


@functools.partial(
    pl.kernel,
    out_type=tuple(jax.ShapeDtypeStruct((ROWS, QP), F32) for _ in range(8)),
    mesh=_sc_mesh(),
    compiler_params=pltpu.CompilerParams(use_tc_tiling_on_sc=False),
    scratch_types=[
        pltpu.VMEM((K, BCH), I32),
        pltpu.VMEM((K, BCH), I32),
        pltpu.VMEM((BCH, QP), F32),
        pltpu.VMEM((BCH, QP), F32),
        pltpu.VMEM((BCH, QP), F32),
        pltpu.VMEM_SHARED((ROWS, QP), F32),
        pltpu.SemaphoreType.DMA,
        pltpu.SemaphoreType.DMA,
    ],
)
def _sc_prep(src_h, hid_h, a0_h, a1_h, a2_h, a3_h, a4_h, a5_h, ones_h, z_h,
             oa0, oa1, oa2, oa3, oa4, oa5, ocn, och,
             srcv, hidv, bufa, bufb, onesb, acc, sema, semb):
    c = lax.axis_index("c")
    s = lax.axis_index("s")

    pltpu.sync_copy(src_h.at[s], srcv)
    pltpu.sync_copy(hid_h.at[s], hidv)
    pltpu.sync_copy(ones_h, onesb)

    ains = (a0_h, a1_h, a2_h, a3_h, a4_h, a5_h)
    aouts = (oa0, oa1, oa2, oa3, oa4, oa5)

    for p in range(3):
        _zero_acc(z_h, acc, s)
        plsc.subcore_barrier()
        _edge_loop(c, ains[2 * p], ains[2 * p + 1], hidv, srcv,
                   bufa, bufb, sema, semb, acc)
        plsc.subcore_barrier()
        _raw_out(c, s, acc, aouts[2 * p], aouts[2 * p + 1])

    _zero_acc(z_h, acc, s)
    plsc.subcore_barrier()

    @pl.when(c == 0)
    def _():
        _scatter_only_loop(srcv, onesb, acc)

    @pl.when(c == 1)
    def _():
        _scatter_only_loop(hidv, onesb, acc)

    plsc.subcore_barrier()
    _raw_out(c, s, acc, ocn, och)


@functools.partial(
    pl.kernel,
    out_type=tuple(jax.ShapeDtypeStruct((ROWS, QL), F32) for _ in range(8))
    + (jax.ShapeDtypeStruct((ZR, QL), F32),),
    mesh=_sc_mesh(),
    compiler_params=pltpu.CompilerParams(use_tc_tiling_on_sc=False),
    scratch_types=[
        pltpu.VMEM((K, BCH), I32),
        pltpu.VMEM((K, BCH), I32),
        pltpu.VMEM((BCH, QL), F32),
        pltpu.VMEM((BCH, QL), F32),
        pltpu.VMEM_SHARED((ROWS, QL), F32),
        pltpu.SemaphoreType.DMA,
        pltpu.SemaphoreType.DMA,
    ],
)
def _sc_seg(gid_h, sid_h, t0_h, t1_h, t2_h, t3_h, t4_h, t5_h, t6_h, t7_h,
            z_h,
            o0, o1, o2, o3, o4, o5, o6, o7, o_z,
            gidv, sidv, bufa, bufb, acc, sema, semb):
    c = lax.axis_index("c")
    s = lax.axis_index("s")

    @pl.when(jnp.logical_and(c == 0, s == 0))
    def _():
        pltpu.sync_copy(z_h, o_z)

    pltpu.sync_copy(gid_h.at[s], gidv)
    pltpu.sync_copy(sid_h.at[s], sidv)

    tins = (t0_h, t1_h, t2_h, t3_h, t4_h, t5_h, t6_h, t7_h)
    outs = (o0, o1, o2, o3, o4, o5, o6, o7)

    for p in range(4):
        _zero_acc(z_h, acc, s)
        plsc.subcore_barrier()
        _edge_loop(c, tins[2 * p], tins[2 * p + 1], gidv, sidv,
                   bufa, bufb, sema, semb, acc)
        plsc.subcore_barrier()
        _raw_out(c, s, acc, outs[2 * p], outs[2 * p + 1])
        plsc.subcore_barrier()


BLK = 512
GRID = ROWS // BLK


def _tc_embed_body(x_ref, w_ref, *h_refs):
    h = jnp.dot(x_ref[...], w_ref[...], preferred_element_type=F32)
    for i in range(8):
        h_refs[i][...] = h[:, 8 * i:8 * i + 8]


def _tc_embed(x_aug, w_aug):
    spec8 = pl.BlockSpec((BLK, QL), lambda i: (i, 0))
    return pl.pallas_call(
        _tc_embed_body,
        grid=(GRID,),
        in_specs=[pl.BlockSpec((BLK, 128), lambda i: (i, 0)),
                  pl.BlockSpec((128, HD), lambda i: (0, 0))],
        out_specs=[spec8] * 8,
        out_shape=[jax.ShapeDtypeStruct((ROWS, QL), F32)] * 8,
    )(x_aug, w_aug)


def _softplus(x):
    return jnp.maximum(x, 0.0) + jnp.log1p(jnp.exp(-jnp.abs(x)))


def _tc_scale_body(*refs):
    s_refs = refs[0:8]
    c_ref = refs[8]
    o_refs = refs[9:17]
    inv = 1.0 / jnp.maximum(c_ref[...][:, 0:1], 1.0)
    for i in range(8):
        o_refs[i][...] = s_refs[i][...] * inv


def _tc_scale(sums, cnt_h):
    spec8 = pl.BlockSpec((BLK, QL), lambda i: (i, 0))
    specc = pl.BlockSpec((BLK, QP), lambda i: (i, 0))
    return tuple(pl.pallas_call(
        _tc_scale_body,
        grid=(GRID,),
        in_specs=[spec8] * 8 + [specc],
        out_specs=[spec8] * 8,
        out_shape=[jax.ShapeDtypeStruct((ROWS, QL), F32)] * 8,
    )(*sums, cnt_h))


def _tc_layer_body(*refs):
    h_refs = refs[0:8]
    a_refs = refs[8:11]
    g_refs = refs[11:19]
    c_ref, w_ref, b_ref = refs[19:22]
    n_refs = refs[22:30]
    cnt = c_ref[...][:, 0:1]
    inv = 1.0 / jnp.maximum(cnt, 1.0)
    m = (cnt > 0.0).astype(F32)
    hq = [r[...] for r in h_refs]
    h = jnp.concatenate(hq, axis=1)
    z = jnp.concatenate(
        [q * m for q in hq]
        + [a[...] * inv for a in a_refs]
        + [g[...] * inv for g in g_refs],
        axis=1)
    o = jnp.dot(z, w_ref[...], preferred_element_type=F32) + b_ref[...]
    zf = o[:, :HD]
    zc = o[:, HD:]
    out = jax.nn.sigmoid(zf) * _softplus(zc)
    hn = _softplus(out + h)
    for i in range(8):
        n_refs[i][...] = hn[:, 8 * i:8 * i + 8]


def _tc_layer(hq, aq, gq, cnt_n, w_stk, b_stk):
    spec8 = pl.BlockSpec((BLK, QL), lambda i: (i, 0))
    specq = pl.BlockSpec((BLK, 16), lambda i: (i, 0))
    specc = pl.BlockSpec((BLK, QP), lambda i: (i, 0))
    return pl.pallas_call(
        _tc_layer_body,
        grid=(GRID,),
        in_specs=[spec8] * 8 + [specq] * 3 + [spec8] * 8 + [
            specc,
            pl.BlockSpec((176, 128), lambda i: (0, 0)),
            pl.BlockSpec((1, 128), lambda i: (0, 0))],
        out_specs=[spec8] * 8,
        out_shape=[jax.ShapeDtypeStruct((ROWS, QL), F32)] * 8,
    )(*hq, *aq, *gq, cnt_n, w_stk, b_stk)


def _tc_out_body(h0_ref, h1_ref, h2_ref, h3_ref, h4_ref, h5_ref, h6_ref,
                 h7_ref, b_ref, wp_ref, bp_ref, wo_ref, bo_ref, o_ref,
                 acc, cacc):
    i = pl.program_id(0)

    @pl.when(i == 0)
    def _():
        acc[...] = jnp.zeros_like(acc)
        cacc[...] = jnp.zeros_like(cacc)

    bb = b_ref[0]
    gid = lax.broadcasted_iota(I32, (GG, BLK), 0)
    maskf = (jnp.broadcast_to(bb, (GG, BLK)) == gid).astype(F32)
    h = jnp.concatenate(
        [h0_ref[...], h1_ref[...], h2_ref[...], h3_ref[...],
         h4_ref[...], h5_ref[...], h6_ref[...], h7_ref[...]], axis=1)
    acc[...] += jnp.dot(maskf, h, preferred_element_type=F32)
    cacc[...] += jnp.broadcast_to(
        jnp.sum(maskf, axis=1, keepdims=True), (GG, 128))

    @pl.when(i == GRID - 1)
    def _():
        cnt = cacc[:, 0:1]
        gm = acc[...] / jnp.maximum(cnt, 1.0)
        p = _softplus(jnp.dot(gm, wp_ref[...], preferred_element_type=F32)
                      + bp_ref[...])
        res = jnp.dot(p, wo_ref[...], preferred_element_type=F32) + bo_ref[...]
        o_ref[...] = res[:, 0:1]


def _tc_out(hq, batch3d, wp, bp, wo8, bo8):
    spec8 = pl.BlockSpec((BLK, QL), lambda i: (i, 0))
    return pl.pallas_call(
        _tc_out_body,
        grid=(GRID,),
        in_specs=[spec8] * 8 + [
            pl.BlockSpec((1, 1, BLK), lambda i: (i, 0, 0)),
            pl.BlockSpec((HD, HOUTD), lambda i: (0, 0)),
            pl.BlockSpec((1, HOUTD), lambda i: (0, 0)),
            pl.BlockSpec((HOUTD, 8), lambda i: (0, 0)),
            pl.BlockSpec((1, 8), lambda i: (0, 0))],
        out_specs=pl.BlockSpec((GG, 1), lambda i: (0, 0)),
        out_shape=jax.ShapeDtypeStruct((GG, 1), F32),
        scratch_shapes=[pltpu.VMEM((GG, HD), F32),
                        pltpu.VMEM((GG, 128), F32)],
    )(*hq, batch3d, wp, bp, wo8, bo8)


def kernel(x, hyperedge_index, hyperedge_attr, batch, W_embed, b_embed,
           Wf, bf, Wc, bc, W_proj, b_proj, W_out, b_out):
    src = hyperedge_index[0].astype(I32)
    hid = hyperedge_index[1].astype(I32)
    padv = jnp.full((EPAD - EE,), PADROW, I32)
    src3 = jnp.concatenate([src, padv]).reshape(NSUB, K, BCH)
    hid3 = jnp.concatenate([hid, padv]).reshape(NSUB, K, BCH)

    zrows_p = jnp.zeros((ZR, QP), F32)
    zrows = jnp.zeros((ZR, QL), F32)
    ones_s = jnp.ones((BCH, QP), F32)

    attr_p = jnp.pad(hyperedge_attr, ((0, ROWS - HH), (0, 48 - HEA)))
    aslices = [attr_p[:, 8 * i:8 * i + 8] for i in range(6)]
    prep = _sc_prep(src3, hid3, *aslices, ones_s, zrows_p)
    a48 = jnp.concatenate(prep[:6], axis=1)
    cnt_n = prep[6]
    cnt_h = prep[7]
    aq = (a48[:, 0:16], a48[:, 16:32], a48[:, 32:48])

    x_aug = jnp.pad(x, ((0, ROWS - NN), (0, 128 - NIN)))
    x_aug = x_aug.at[:, NIN].set(1.0)
    w_aug = jnp.zeros((128, HD), F32).at[:NIN].set(W_embed).at[NIN].set(b_embed)
    hq = tuple(_tc_embed(x_aug, w_aug))

    def stack_w(wf_i, wc_i):
        return jnp.concatenate([
            jnp.concatenate([wf_i[:96], wc_i[:96]], 1),
            jnp.pad(jnp.concatenate([wf_i[96:99], wc_i[96:99]], 1),
                    ((0, 13), (0, 0))),
            jnp.concatenate([wf_i[99:], wc_i[99:]], 1),
        ], 0)

    ws = jnp.stack([stack_w(Wf[i], Wc[i]) for i in range(LL)])
    bs = jnp.stack([jnp.concatenate([bf[i], bc[i]])[None, :]
                    for i in range(LL)])

    def half_step(t, carry):
        hq_c, tab, ztok = carry
        even = (t % 2) == 0
        gids = jnp.where(even, src3, hid3)
        sids = jnp.where(even, hid3, src3)
        outs = _sc_seg(gids, sids, *tab, ztok)
        sums = tuple(outs[:8])

        def do_scale(_):
            return hq_c, _tc_scale(sums, cnt_h)

        def do_gate(_):
            w_stk = lax.dynamic_index_in_dim(ws, t // 2, keepdims=False)
            b_stk = lax.dynamic_index_in_dim(bs, t // 2, keepdims=False)
            hn = tuple(_tc_layer(hq_c, aq, sums, cnt_n, w_stk, b_stk))
            return hn, hn

        hq_n, tab_n = lax.cond(even, do_scale, do_gate, 0)
        return hq_n, tab_n, outs[8]

    nsteps = lax.optimization_barrier(jnp.int32(2 * LL))
    hq, _, _ = lax.fori_loop(0, nsteps, half_step, (hq, hq, zrows))

    batch3d = jnp.pad(batch.astype(I32), (0, ROWS - NN),
                      constant_values=GG).reshape(GRID, 1, BLK)
    wo8 = jnp.pad(W_out, ((0, 0), (0, 7)))
    bo8 = jnp.pad(b_out, (0, 7))[None, :]
    return _tc_out(hq, batch3d, W_proj, b_proj[None, :], wo8, bo8)
